# baseline (speedup 1.0000x reference)
import functools

import jax
import jax.numpy as jnp
from jax import lax
from jax.experimental import pallas as pl
from jax.experimental.pallas import tpu as pltpu
from jax.experimental.pallas import tpu_sc as plsc

P = 262144
NN = 262144
W = 8
ALPHA = 0.99
DT = 1e-3
SMALL = 1e-10

B = 2048
NB = P // B
NSUB = 16
PT = P // NSUB
NT = NN // NSUB
FCH = 2048
CH = 8192


def _k1_body(pv_ref, vf_ref, st_ref, sf_ref, sgj_ref, e_ref):
    pv = pv_ref[...]
    vf = vf_ref[...]
    st = st_ref[...]
    sf = sf_ref[...]
    sgj = sgj_ref[...]
    m = pv[0:1]
    vol = pv[1:2]
    sm = sf * m
    mom = [sm * vf[ci:ci + 1] for ci in range(3)]
    momnt = []
    for ci in range(3):
        sif = (st[3 * ci + 0:3 * ci + 1] * sgj[0:8]
               + st[3 * ci + 1:3 * ci + 2] * sgj[8:16]
               + st[3 * ci + 2:3 * ci + 3] * sgj[16:24])
        sif = -vol * sif
        momnt.append(mom[ci] + DT * (sif + sf * vf[3 + ci:4 + ci]))
    e_ref[...] = jnp.concatenate([sm] + mom + momnt, axis=0)


def _k1(pv, vf, st, sf, sgj):
    return pl.pallas_call(
        _k1_body,
        grid=(NB,),
        in_specs=[
            pl.BlockSpec((2, B), lambda i: (0, i)),
            pl.BlockSpec((6, B), lambda i: (0, i)),
            pl.BlockSpec((9, B), lambda i: (0, i)),
            pl.BlockSpec((8, B), lambda i: (0, i)),
            pl.BlockSpec((24, B), lambda i: (0, i)),
        ],
        out_specs=pl.BlockSpec((56, B), lambda i: (0, i)),
        out_shape=jax.ShapeDtypeStruct((56, P), jnp.float32),
    )(pv, vf, st, sf, sgj)


def _ksc_body(e, connr, z, acca, accb, ga, gb,
              acc0, acc1, acc2, acc3, val3, idx2, g2,
              misc_s, idx_s, vals0, vals1, vals2, scat0, scat1, scat2,
              gat0, gat1, gout0, gout1):
    c = lax.axis_index("c")
    s = lax.axis_index("s")
    p0 = s * PT
    n0 = s * NT
    q0 = s * (PT // 128)
    accs = (acc0, acc1, acc2, acc3)
    vals = (vals0, vals1, vals2)
    scat = (scat0, scat1, scat2)
    gat = (gat0, gat1)
    gout = (gout0, gout1)

    def drain_ch(sem):
        pltpu.make_async_copy(e.at[0, pl.ds(0, CH)], g2.at[pl.ds(0, CH)], sem).wait()

    def drain_idx():
        pltpu.make_async_copy(connr.at[0, pl.ds(0, 128)], idx2, idx_s).wait()

    def core_work(rows_map, acc_out, g_out):
        zd = [pltpu.async_copy(z, accs[k].at[pl.ds(n0, NT)], misc_s)
              for k in range(4)]
        pltpu.async_copy(e.at[rows_map(0, 0), pl.ds(p0, CH)], val3.at[pl.ds(0, CH)],
                         vals[0])
        pltpu.async_copy(connr.at[0, pl.ds(q0, PT // 128)], idx2, idx_s)
        for d in zd:
            d.wait()
        plsc.subcore_barrier()

        pend = []
        for sr in range(64):
            r, h = divmod(sr, 2)
            w, cc = divmod(r, 4)
            b = sr % 3
            while len(pend) > 1:
                drain_ch(scat[pend.pop(0)])
            if sr + 1 < 64:
                r2, h2 = divmod(sr + 1, 2)
                w2, cc2 = divmod(r2, 4)
                pltpu.async_copy(
                    e.at[rows_map(cc2, w2), pl.ds(p0 + h2 * CH, CH)],
                    val3.at[pl.ds(((sr + 1) % 3) * CH, CH)], vals[(sr + 1) % 3])
            drain_ch(vals[b])
            if sr % 8 == 0:
                while pend:
                    drain_ch(scat[pend.pop(0)])
                if sr > 0:
                    pltpu.async_copy(connr.at[w, pl.ds(q0, PT // 128)],
                                     idx2, idx_s)
                drain_idx()

            @plsc.parallel_loop(0, CH // 128, unroll=4)
            def sbody(j, _cc=cc, _h=h, _b=b):
                pltpu.async_copy(val3.at[pl.ds(_b * CH + j * 128, 128)],
                                 accs[_cc].at[idx2.at[_h * (CH // 128) + j]],
                                 scat[_b], add=True)
            pend.append(b)
        while pend:
            drain_ch(scat[pend.pop(0)])
        plsc.subcore_barrier()

        od = [pltpu.async_copy(accs[k].at[pl.ds(n0, NT)],
                               acc_out.at[k, pl.ds(n0, NT)], misc_s)
              for k in range(4)]
        for d in od:
            d.wait()
        fslot = (0, FCH, 2 * FCH, 3 * FCH)
        for ch in range(NT // FCH):
            nb = n0 + ch * FCH
            ld = [pltpu.async_copy(accs[k].at[pl.ds(nb, FCH)],
                                   g2.at[pl.ds(fslot[k], FCH)], misc_s)
                  for k in range(4)]
            for d in ld:
                d.wait()

            def fbody(k, _):
                mvec = g2[pl.ds(k * 16, 16)]
                valid = mvec > SMALL
                safe = jnp.where(valid, mvec, 1.0)
                for fs in fslot[1:]:
                    v = g2[pl.ds(fs + k * 16, 16)]
                    g2[pl.ds(fs + k * 16, 16)] = jnp.where(
                        valid, v / safe, 0.0)
                return 0
            lax.fori_loop(0, FCH // 16, fbody, 0)
            sd = [pltpu.async_copy(g2.at[pl.ds(fslot[1 + k], FCH)],
                                   accs[1 + k].at[pl.ds(nb, FCH)], misc_s)
                  for k in range(3)]
            for d in sd:
                d.wait()
        plsc.subcore_barrier()

        subs = [(w, cc, h) for w in range(W) for cc in range(1, 4)
                for h in range(2)]
        pltpu.async_copy(connr.at[0, pl.ds(q0, PT // 128)], idx2, idx_s)
        pendg = []

        def flush_gather():
            while pendg:
                pb, pw, pcc, ph = pendg.pop(0)
                drain_ch(gat[pb])
                pltpu.async_copy(
                    g2.at[pl.ds(pb * CH, CH)],
                    g_out.at[(pcc - 1) * 8 + pw, pl.ds(p0 + ph * CH, CH)],
                    gout[pb])
        for sg, (w, cc, h) in enumerate(subs):
            b = sg % 2
            if sg >= 2:
                drain_ch(gout[b])
            if sg % 6 == 0:
                flush_gather()
                if sg > 0:
                    pltpu.async_copy(connr.at[w, pl.ds(q0, PT // 128)],
                                     idx2, idx_s)
                drain_idx()

            @plsc.parallel_loop(0, CH // 128, unroll=4)
            def gbody(j, _cc=cc, _h=h, _b=b):
                pltpu.async_copy(accs[_cc].at[idx2.at[_h * (CH // 128) + j]],
                                 g2.at[pl.ds(_b * CH + j * 128, 128)], gat[_b])
            pendg.append((b, w, cc, h))
            if len(pendg) > 1:
                pb, pw, pcc, ph = pendg.pop(0)
                drain_ch(gat[pb])
                pltpu.async_copy(
                    g2.at[pl.ds(pb * CH, CH)],
                    g_out.at[(pcc - 1) * 8 + pw, pl.ds(p0 + ph * CH, CH)],
                    gout[pb])
        flush_gather()
        for b in (0, 1):
            drain_ch(gout[b])

    @pl.when(c == 0)
    def _():
        core_work(lambda cc, w: cc * 8 + w, acca, ga)

    @pl.when(c == 1)
    def _():
        core_work(lambda cc, w: w if cc == 0 else 32 + (cc - 1) * 8 + w,
                  accb, gb)


def _ksc(e, connr, z):
    mesh = plsc.VectorSubcoreMesh(core_axis_name="c", subcore_axis_name="s")
    f = functools.partial(
        pl.kernel,
        out_type=[
            jax.ShapeDtypeStruct((4, NN), jnp.float32),
            jax.ShapeDtypeStruct((4, NN), jnp.float32),
            jax.ShapeDtypeStruct((24, P), jnp.float32),
            jax.ShapeDtypeStruct((24, P), jnp.float32),
        ],
        mesh=mesh,
        scratch_types=[
            pltpu.VMEM_SHARED((NN,), jnp.float32),
            pltpu.VMEM_SHARED((NN,), jnp.float32),
            pltpu.VMEM_SHARED((NN,), jnp.float32),
            pltpu.VMEM_SHARED((NN,), jnp.float32),
            pltpu.VMEM((3 * CH,), jnp.float32),
            pltpu.VMEM((128, 128), jnp.int32),
            pltpu.VMEM((2 * CH,), jnp.float32),
        ] + [pltpu.SemaphoreType.DMA] * 12,
    )(_ksc_body)
    return f(e, connr, z)


def _k5_body(ga_ref, gb_ref, sf_ref, sgj_ref, vp_ref, ft_ref,
             nvol_ref, nf_ref, lp_ref, npos_ref, nvel_ref):
    ga = ga_ref[...]
    gb = gb_ref[...]
    sf = sf_ref[...]
    sgj = sgj_ref[...]
    vp = vp_ref[...]
    ft = ft_ref[...]
    vel = [ga[8 * ci:8 * ci + 8] for ci in range(3)]
    velnt = [gb[8 * ci:8 * ci + 8] for ci in range(3)]
    dvp3 = jnp.concatenate(
        [jnp.sum(sf * (velnt[ci] - vel[ci]), axis=0, keepdims=True)
         for ci in range(3)], axis=0)
    vnp3 = jnp.concatenate(
        [jnp.sum(sf * velnt[ci], axis=0, keepdims=True) for ci in range(3)],
        axis=0)
    l9 = jnp.concatenate(
        [jnp.sum(sgj[8 * i:8 * i + 8] * velnt[j], axis=0, keepdims=True)
         for i in range(3) for j in range(3)], axis=0)
    nvel3 = (1.0 - ALPHA) * vnp3 + ALPHA * (vp[0:3] + dvp3)
    npos3 = vp[3:6] + vnp3 * DT
    row = lax.broadcasted_iota(jnp.int32, (9, 1), 0)
    eye9 = jnp.where((row == 0) | (row == 4) | (row == 8), 1.0, 0.0)
    a9 = eye9 + l9 * DT
    nf9 = None
    for k in range(3):
        ak = jnp.concatenate([a9[k:k + 1]] * 3 + [a9[3 + k:4 + k]] * 3
                             + [a9[6 + k:7 + k]] * 3, axis=0)
        fk = jnp.concatenate([ft[3 * k:3 * k + 3]] * 3, axis=0)
        nf9 = ak * fk if nf9 is None else nf9 + ak * fk
    xa = jnp.concatenate([nf9[4:5], nf9[5:6], nf9[3:4]], axis=0)
    xb = jnp.concatenate([nf9[8:9], nf9[6:7], nf9[7:8]], axis=0)
    xc = jnp.concatenate([nf9[5:6], nf9[3:4], nf9[4:5]], axis=0)
    xd = jnp.concatenate([nf9[7:8], nf9[8:9], nf9[6:7]], axis=0)
    cof = xa * xb - xc * xd
    det = jnp.sum(nf9[0:3] * cof, axis=0, keepdims=True)
    nvol_ref[...] = det * vp[6:7]
    nf_ref[...] = nf9
    lp_ref[...] = l9
    npos_ref[...] = npos3
    nvel_ref[...] = nvel3


def _k5(ga, gb, sf, sgj, vp, ft):
    return pl.pallas_call(
        _k5_body,
        grid=(NB,),
        in_specs=[
            pl.BlockSpec((24, B), lambda i: (0, i)),
            pl.BlockSpec((24, B), lambda i: (0, i)),
            pl.BlockSpec((8, B), lambda i: (0, i)),
            pl.BlockSpec((24, B), lambda i: (0, i)),
            pl.BlockSpec((7, B), lambda i: (0, i)),
            pl.BlockSpec((9, B), lambda i: (0, i)),
        ],
        out_specs=[
            pl.BlockSpec((1, B), lambda i: (0, i)),
            pl.BlockSpec((9, B), lambda i: (0, i)),
            pl.BlockSpec((9, B), lambda i: (0, i)),
            pl.BlockSpec((3, B), lambda i: (0, i)),
            pl.BlockSpec((3, B), lambda i: (0, i)),
        ],
        out_shape=[
            jax.ShapeDtypeStruct((1, P), jnp.float32),
            jax.ShapeDtypeStruct((9, P), jnp.float32),
            jax.ShapeDtypeStruct((9, P), jnp.float32),
            jax.ShapeDtypeStruct((3, P), jnp.float32),
            jax.ShapeDtypeStruct((3, P), jnp.float32),
        ],
    )(ga, gb, sf, sgj, vp, ft)


def kernel(mass, volume, volume0, velocity, force, stress, position, F,
           shapef, shapef_grad, conn):
    velT = velocity.T
    pv = jnp.stack([mass, volume])
    vf = jnp.concatenate([velT, force.T], axis=0)
    st = stress.reshape(P, 9).T
    sf = shapef.T
    sgj = shapef_grad.transpose(2, 1, 0).reshape(24, P)
    connr = conn.T.reshape(W, P // 128, 128)
    z = jnp.zeros((NT,), jnp.float32)
    vp = jnp.concatenate([velT, position.T, volume0[None, :]], axis=0)
    ft = F.reshape(P, 9).T

    e = _k1(pv, vf, st, sf, sgj)
    acca, accb, ga, gb = _ksc(e, connr, z)
    nvol, nf, lp, npos, nvel = _k5(ga, gb, sf, sgj, vp, ft)

    next_vol = nvol[0]
    next_F = nf.T.reshape(P, 3, 3)
    L_p = lp.T.reshape(P, 3, 3)
    next_pos = npos.T
    next_vel = nvel.T
    node_mass = acca[0]
    node_moment = acca[1:4].T
    node_moment_nt = accb[1:4].T
    return (next_vol, next_F, L_p, next_pos, next_vel,
            node_mass, node_moment, node_moment_nt)


# speedup vs baseline: 1.0107x; 1.0107x over previous
import functools

import jax
import jax.numpy as jnp
from jax import lax
from jax.experimental import pallas as pl
from jax.experimental.pallas import tpu as pltpu
from jax.experimental.pallas import tpu_sc as plsc

P = 262144
NN = 262144
W = 8
ALPHA = 0.99
DT = 1e-3
SMALL = 1e-10

B = 2048
NB = P // B
NSUB = 16
PT = P // NSUB
NT = NN // NSUB
FCH = 2048
CH = 8192


def _k1_body(pv_ref, vf_ref, st_ref, sf_ref, sgj_ref, e_ref):
    pv = pv_ref[...]
    vf = vf_ref[...]
    st = st_ref[...]
    sf = sf_ref[...]
    sgj = sgj_ref[...]
    m = pv[0:1]
    vol = pv[1:2]
    sm = sf * m
    mom = [sm * vf[ci:ci + 1] for ci in range(3)]
    momnt = []
    for ci in range(3):
        sif = (st[3 * ci + 0:3 * ci + 1] * sgj[0:8]
               + st[3 * ci + 1:3 * ci + 2] * sgj[8:16]
               + st[3 * ci + 2:3 * ci + 3] * sgj[16:24])
        sif = -vol * sif
        momnt.append(mom[ci] + DT * (sif + sf * vf[3 + ci:4 + ci]))
    e_ref[...] = jnp.concatenate([sm] + mom + momnt, axis=0)


def _k1(pv, vf, st, sf, sgj):
    return pl.pallas_call(
        _k1_body,
        grid=(NB,),
        in_specs=[
            pl.BlockSpec((2, B), lambda i: (0, i)),
            pl.BlockSpec((6, B), lambda i: (0, i)),
            pl.BlockSpec((9, B), lambda i: (0, i)),
            pl.BlockSpec((8, B), lambda i: (0, i)),
            pl.BlockSpec((24, B), lambda i: (0, i)),
        ],
        out_specs=pl.BlockSpec((56, B), lambda i: (0, i)),
        out_shape=jax.ShapeDtypeStruct((56, P), jnp.float32),
    )(pv, vf, st, sf, sgj)


def _ksc_body(e, connr, z, acca, accb, ga, gb,
              acc0, acc1, acc2, acc3, val3, idx2, g2,
              misc_s, idx_s, vals0, vals1, vals2, scat0, scat1, scat2,
              gat0, gat1, gout0, gout1):
    c = lax.axis_index("c")
    s = lax.axis_index("s")
    p0 = s * PT
    n0 = s * NT
    q0 = s * (PT // 128)
    accs = (acc0, acc1, acc2, acc3)
    vals = (vals0, vals1, vals2)
    scat = (scat0, scat1, scat2)
    gat = (gat0, gat1)
    gout = (gout0, gout1)

    def drain_ch(sem):
        pltpu.make_async_copy(e.at[0, pl.ds(0, CH)], g2.at[pl.ds(0, CH)], sem).wait()

    def drain_idx():
        pltpu.make_async_copy(connr.at[0, pl.ds(0, 128)], idx2, idx_s).wait()

    def core_work(rows_map, acc_out, g_out):
        zd = [pltpu.async_copy(z, accs[k].at[pl.ds(n0, NT)], misc_s)
              for k in range(4)]
        pltpu.async_copy(e.at[rows_map(0, 0), pl.ds(p0, CH)], val3.at[pl.ds(0, CH)],
                         vals[0])
        pltpu.async_copy(connr.at[0, pl.ds(q0, PT // 128)], idx2, idx_s)
        for d in zd:
            d.wait()
        plsc.subcore_barrier()

        pend = []
        for sr in range(64):
            r, h = divmod(sr, 2)
            w, cc = divmod(r, 4)
            b = sr % 3
            while len(pend) > 1:
                drain_ch(scat[pend.pop(0)])
            if sr + 1 < 64:
                r2, h2 = divmod(sr + 1, 2)
                w2, cc2 = divmod(r2, 4)
                pltpu.async_copy(
                    e.at[rows_map(cc2, w2), pl.ds(p0 + h2 * CH, CH)],
                    val3.at[pl.ds(((sr + 1) % 3) * CH, CH)], vals[(sr + 1) % 3])
            drain_ch(vals[b])
            if sr % 8 == 0:
                while pend:
                    drain_ch(scat[pend.pop(0)])
                if sr > 0:
                    pltpu.async_copy(connr.at[w, pl.ds(q0, PT // 128)],
                                     idx2, idx_s)
                drain_idx()

            def sbody(j, _, _cc=cc, _h=h, _b=b):
                pltpu.async_copy(val3.at[pl.ds(_b * CH + j * 128, 128)],
                                 accs[_cc].at[idx2.at[_h * (CH // 128) + j]],
                                 scat[_b], add=True)
                return 0
            lax.fori_loop(0, CH // 128, sbody, 0)
            pend.append(b)
        while pend:
            drain_ch(scat[pend.pop(0)])
        plsc.subcore_barrier()

        od = [pltpu.async_copy(accs[k].at[pl.ds(n0, NT)],
                               acc_out.at[k, pl.ds(n0, NT)], misc_s)
              for k in range(4)]
        for d in od:
            d.wait()
        fslot = (0, FCH, 2 * FCH, 3 * FCH)
        for ch in range(NT // FCH):
            nb = n0 + ch * FCH
            ld = [pltpu.async_copy(accs[k].at[pl.ds(nb, FCH)],
                                   g2.at[pl.ds(fslot[k], FCH)], misc_s)
                  for k in range(4)]
            for d in ld:
                d.wait()

            def fbody(k, _):
                mvec = g2[pl.ds(k * 16, 16)]
                valid = mvec > SMALL
                safe = jnp.where(valid, mvec, 1.0)
                for fs in fslot[1:]:
                    v = g2[pl.ds(fs + k * 16, 16)]
                    g2[pl.ds(fs + k * 16, 16)] = jnp.where(
                        valid, v / safe, 0.0)
                return 0
            lax.fori_loop(0, FCH // 16, fbody, 0)
            sd = [pltpu.async_copy(g2.at[pl.ds(fslot[1 + k], FCH)],
                                   accs[1 + k].at[pl.ds(nb, FCH)], misc_s)
                  for k in range(3)]
            for d in sd:
                d.wait()
        plsc.subcore_barrier()

        subs = [(w, cc, h) for w in range(W) for cc in range(1, 4)
                for h in range(2)]
        pltpu.async_copy(connr.at[0, pl.ds(q0, PT // 128)], idx2, idx_s)
        pendg = []

        def flush_gather():
            while pendg:
                pb, pw, pcc, ph = pendg.pop(0)
                drain_ch(gat[pb])
                pltpu.async_copy(
                    g2.at[pl.ds(pb * CH, CH)],
                    g_out.at[(pcc - 1) * 8 + pw, pl.ds(p0 + ph * CH, CH)],
                    gout[pb])
        for sg, (w, cc, h) in enumerate(subs):
            b = sg % 2
            if sg >= 2:
                drain_ch(gout[b])
            if sg % 6 == 0:
                flush_gather()
                if sg > 0:
                    pltpu.async_copy(connr.at[w, pl.ds(q0, PT // 128)],
                                     idx2, idx_s)
                drain_idx()

            def gbody(j, _, _cc=cc, _h=h, _b=b):
                pltpu.async_copy(accs[_cc].at[idx2.at[_h * (CH // 128) + j]],
                                 g2.at[pl.ds(_b * CH + j * 128, 128)], gat[_b])
                return 0
            lax.fori_loop(0, CH // 128, gbody, 0)
            pendg.append((b, w, cc, h))
            if len(pendg) > 1:
                pb, pw, pcc, ph = pendg.pop(0)
                drain_ch(gat[pb])
                pltpu.async_copy(
                    g2.at[pl.ds(pb * CH, CH)],
                    g_out.at[(pcc - 1) * 8 + pw, pl.ds(p0 + ph * CH, CH)],
                    gout[pb])
        flush_gather()
        for b in (0, 1):
            drain_ch(gout[b])

    @pl.when(c == 0)
    def _():
        core_work(lambda cc, w: cc * 8 + w, acca, ga)

    @pl.when(c == 1)
    def _():
        core_work(lambda cc, w: w if cc == 0 else 32 + (cc - 1) * 8 + w,
                  accb, gb)


def _ksc(e, connr, z):
    mesh = plsc.VectorSubcoreMesh(core_axis_name="c", subcore_axis_name="s")
    f = functools.partial(
        pl.kernel,
        out_type=[
            jax.ShapeDtypeStruct((4, NN), jnp.float32),
            jax.ShapeDtypeStruct((4, NN), jnp.float32),
            jax.ShapeDtypeStruct((24, P), jnp.float32),
            jax.ShapeDtypeStruct((24, P), jnp.float32),
        ],
        mesh=mesh,
        scratch_types=[
            pltpu.VMEM_SHARED((NN,), jnp.float32),
            pltpu.VMEM_SHARED((NN,), jnp.float32),
            pltpu.VMEM_SHARED((NN,), jnp.float32),
            pltpu.VMEM_SHARED((NN,), jnp.float32),
            pltpu.VMEM((3 * CH,), jnp.float32),
            pltpu.VMEM((128, 128), jnp.int32),
            pltpu.VMEM((2 * CH,), jnp.float32),
        ] + [pltpu.SemaphoreType.DMA] * 12,
    )(_ksc_body)
    return f(e, connr, z)


def _k5_body(ga_ref, gb_ref, sf_ref, sgj_ref, vp_ref, ft_ref,
             nvol_ref, nf_ref, lp_ref, npos_ref, nvel_ref):
    ga = ga_ref[...]
    gb = gb_ref[...]
    sf = sf_ref[...]
    sgj = sgj_ref[...]
    vp = vp_ref[...]
    ft = ft_ref[...]
    vel = [ga[8 * ci:8 * ci + 8] for ci in range(3)]
    velnt = [gb[8 * ci:8 * ci + 8] for ci in range(3)]
    dvp3 = jnp.concatenate(
        [jnp.sum(sf * (velnt[ci] - vel[ci]), axis=0, keepdims=True)
         for ci in range(3)], axis=0)
    vnp3 = jnp.concatenate(
        [jnp.sum(sf * velnt[ci], axis=0, keepdims=True) for ci in range(3)],
        axis=0)
    l9 = jnp.concatenate(
        [jnp.sum(sgj[8 * i:8 * i + 8] * velnt[j], axis=0, keepdims=True)
         for i in range(3) for j in range(3)], axis=0)
    nvel3 = (1.0 - ALPHA) * vnp3 + ALPHA * (vp[0:3] + dvp3)
    npos3 = vp[3:6] + vnp3 * DT
    row = lax.broadcasted_iota(jnp.int32, (9, 1), 0)
    eye9 = jnp.where((row == 0) | (row == 4) | (row == 8), 1.0, 0.0)
    a9 = eye9 + l9 * DT
    nf9 = None
    for k in range(3):
        ak = jnp.concatenate([a9[k:k + 1]] * 3 + [a9[3 + k:4 + k]] * 3
                             + [a9[6 + k:7 + k]] * 3, axis=0)
        fk = jnp.concatenate([ft[3 * k:3 * k + 3]] * 3, axis=0)
        nf9 = ak * fk if nf9 is None else nf9 + ak * fk
    xa = jnp.concatenate([nf9[4:5], nf9[5:6], nf9[3:4]], axis=0)
    xb = jnp.concatenate([nf9[8:9], nf9[6:7], nf9[7:8]], axis=0)
    xc = jnp.concatenate([nf9[5:6], nf9[3:4], nf9[4:5]], axis=0)
    xd = jnp.concatenate([nf9[7:8], nf9[8:9], nf9[6:7]], axis=0)
    cof = xa * xb - xc * xd
    det = jnp.sum(nf9[0:3] * cof, axis=0, keepdims=True)
    nvol_ref[...] = det * vp[6:7]
    nf_ref[...] = nf9
    lp_ref[...] = l9
    npos_ref[...] = npos3
    nvel_ref[...] = nvel3


def _k5(ga, gb, sf, sgj, vp, ft):
    return pl.pallas_call(
        _k5_body,
        grid=(NB,),
        in_specs=[
            pl.BlockSpec((24, B), lambda i: (0, i)),
            pl.BlockSpec((24, B), lambda i: (0, i)),
            pl.BlockSpec((8, B), lambda i: (0, i)),
            pl.BlockSpec((24, B), lambda i: (0, i)),
            pl.BlockSpec((7, B), lambda i: (0, i)),
            pl.BlockSpec((9, B), lambda i: (0, i)),
        ],
        out_specs=[
            pl.BlockSpec((1, B), lambda i: (0, i)),
            pl.BlockSpec((9, B), lambda i: (0, i)),
            pl.BlockSpec((9, B), lambda i: (0, i)),
            pl.BlockSpec((3, B), lambda i: (0, i)),
            pl.BlockSpec((3, B), lambda i: (0, i)),
        ],
        out_shape=[
            jax.ShapeDtypeStruct((1, P), jnp.float32),
            jax.ShapeDtypeStruct((9, P), jnp.float32),
            jax.ShapeDtypeStruct((9, P), jnp.float32),
            jax.ShapeDtypeStruct((3, P), jnp.float32),
            jax.ShapeDtypeStruct((3, P), jnp.float32),
        ],
    )(ga, gb, sf, sgj, vp, ft)


def kernel(mass, volume, volume0, velocity, force, stress, position, F,
           shapef, shapef_grad, conn):
    velT = velocity.T
    pv = jnp.stack([mass, volume])
    vf = jnp.concatenate([velT, force.T], axis=0)
    st = stress.reshape(P, 9).T
    sf = shapef.T
    sgj = shapef_grad.transpose(2, 1, 0).reshape(24, P)
    connr = conn.T.reshape(W, P // 128, 128)
    z = jnp.zeros((NT,), jnp.float32)
    vp = jnp.concatenate([velT, position.T, volume0[None, :]], axis=0)
    ft = F.reshape(P, 9).T

    e = _k1(pv, vf, st, sf, sgj)
    acca, accb, ga, gb = _ksc(e, connr, z)
    nvol, nf, lp, npos, nvel = _k5(ga, gb, sf, sgj, vp, ft)

    next_vol = nvol[0]
    next_F = nf.T.reshape(P, 3, 3)
    L_p = lp.T.reshape(P, 3, 3)
    next_pos = npos.T
    next_vel = nvel.T
    node_mass = acca[0]
    node_moment = acca[1:4].T
    node_moment_nt = accb[1:4].T
    return (next_vol, next_F, L_p, next_pos, next_vel,
            node_mass, node_moment, node_moment_nt)
